# baseline (speedup 1.0000x reference)
import jax
import jax.numpy as jnp
from jax import lax
from jax.experimental import pallas as pl
from jax.experimental.pallas import tpu as pltpu
from jax.experimental.pallas import tpu_sc as plsc

_B, _C = 4, 3
_GRID = 33
_NPIX = 512 * 512
_NLUT = _GRID * _GRID * _GRID
_NLUT_PAD = 35944
_NC, _NS = 2, 16
_NW = _NC * _NS
_TPB = _NW // _B
_PPT = _NPIX // _TPB
_CHUNK = 512
_NCHUNK = _PPT // _CHUNK
_L = 16


_CMAX = 31.999998092651367
_SHIFTS = (0, 1, _GRID, _GRID + 1,
           _GRID * _GRID, _GRID * _GRID + 1,
           _GRID * _GRID + _GRID, _GRID * _GRID + _GRID + 1)
_SLICE = 34816


def _sc_body(img_hbm, lut_hbm, out_hbm,
             lut0, lut1, lut2,
             xa, ya, za, oa0, oa1, oa2,
             xb, yb, zb, ob0, ob1, ob2,
             sem_ia, sem_ib, sem_oa, sem_ob, sem_lut):
    wid = lax.axis_index("s") * _NC + lax.axis_index("c")
    b = wid // _TPB
    sub = wid % _TPB
    lbase = b * (_C * _NLUT_PAD)
    pltpu.async_copy(lut_hbm.at[pl.ds(pl.multiple_of(lbase, 8), _NLUT_PAD)], lut0, sem_lut)
    pltpu.async_copy(lut_hbm.at[pl.ds(pl.multiple_of(lbase + _NLUT_PAD, 8), _NLUT_PAD)], lut1, sem_lut)
    pltpu.async_copy(lut_hbm.at[pl.ds(pl.multiple_of(lbase + 2 * _NLUT_PAD, 8), _NLUT_PAD)], lut2, sem_lut)
    base = b * (_C * _NPIX) + sub * _PPT

    set_a = (xa, ya, za, oa0, oa1, oa2, sem_ia, sem_oa)
    set_b = (xb, yb, zb, ob0, ob1, ob2, sem_ib, sem_ob)

    def start_in(bufs, t):
        x, y, z, _, _, _, sem_i, _ = bufs
        off = pl.multiple_of(base + t * _CHUNK, _CHUNK)
        pltpu.async_copy(img_hbm.at[pl.ds(off, _CHUNK)], x, sem_i)
        pltpu.async_copy(img_hbm.at[pl.ds(off + _NPIX, _CHUNK)], y, sem_i)
        pltpu.async_copy(img_hbm.at[pl.ds(off + 2 * _NPIX, _CHUNK)], z, sem_i)

    def wait_in(bufs):
        x, y, z, _, _, _, sem_i, _ = bufs
        for d in (x, y, z):
            pltpu.make_async_copy(img_hbm.at[pl.ds(0, _CHUNK)], d, sem_i).wait()

    def start_out(bufs, t):
        _, _, _, p0, p1, p2, _, sem_o = bufs
        off = pl.multiple_of(base + t * _CHUNK, _CHUNK)
        pltpu.async_copy(p0, out_hbm.at[pl.ds(off, _CHUNK)], sem_o)
        pltpu.async_copy(p1, out_hbm.at[pl.ds(off + _NPIX, _CHUNK)], sem_o)
        pltpu.async_copy(p2, out_hbm.at[pl.ds(off + 2 * _NPIX, _CHUNK)], sem_o)

    def wait_out(bufs):
        _, _, _, p0, p1, p2, _, sem_o = bufs
        for s in (p0, p1, p2):
            pltpu.make_async_copy(s, out_hbm.at[pl.ds(0, _CHUNK)], sem_o).wait()

    def compute(bufs):
        x_r, y_r, z_r, p0, p1, p2, _, _ = bufs

        @plsc.parallel_loop(0, _CHUNK, step=_L, unroll=2)
        def _grp(g):
            s = pl.multiple_of(g, _L)

            def coord(v):
                cc = jnp.minimum(jnp.maximum(v * (_GRID - 1.0), 0.0), _CMAX)
                i0 = cc.astype(jnp.int32)
                w = cc - i0.astype(jnp.float32)
                return i0, w

            x0, wx = coord(x_r[pl.ds(s, _L)])
            y0, wy = coord(y_r[pl.ds(s, _L)])
            z0, wz = coord(z_r[pl.ds(s, _L)])
            i000 = (z0 * _GRID + y0) * _GRID + x0
            idx = [i000 + k if k else i000 for k in _SHIFTS]

            ux = 1.0 - wx
            w11 = wx * wy
            w10 = wy - w11
            w01 = wx - w11
            w00 = ux - w10

            corners = [[plsc.load_gather(ref, [idx[k]]) for k in range(8)]
                       for ref in (lut0, lut1, lut2)]
            for (c000, c001, c010, c011, c100, c101, c110, c111), ob in zip(
                    corners, (p0, p1, p2)):
                r0 = (w00 * c000 + w01 * c001) + (w10 * c010 + w11 * c011)
                r1 = (w00 * c100 + w01 * c101) + (w10 * c110 + w11 * c111)
                ob[pl.ds(s, _L)] = r0 + wz * (r1 - r0)

    start_in(set_a, 0)
    for d in (lut0, lut1, lut2):
        pltpu.make_async_copy(lut_hbm.at[pl.ds(0, _NLUT_PAD)], d, sem_lut).wait()

    @pl.loop(0, _NCHUNK, step=2)
    def _pair(t):
        start_in(set_b, t + 1)
        wait_in(set_a)

        @pl.when(t >= 2)
        def _():
            wait_out(set_a)

        compute(set_a)
        start_out(set_a, t)

        @pl.when(t + 2 < _NCHUNK)
        def _():
            start_in(set_a, t + 2)

        wait_in(set_b)

        @pl.when(t >= 2)
        def _():
            wait_out(set_b)

        compute(set_b)
        start_out(set_b, t + 1)

    wait_out(set_a)
    wait_out(set_b)


def kernel(img, lut):
    imgf = img.reshape(_B * _C * _NPIX)
    lutf = lut.reshape(_B, _C, _NLUT)
    lutp = jnp.pad(lutf, ((0, 0), (0, 0), (0, _NLUT_PAD - _NLUT)))
    lutp = lutp.reshape(_B * _C * _NLUT_PAD)
    vm = lambda n: pltpu.VMEM((n,), jnp.float32)
    k = pl.kernel(
        _sc_body,
        out_type=jax.ShapeDtypeStruct((_B * _C * _NPIX,), jnp.float32),
        mesh=plsc.VectorSubcoreMesh(core_axis_name="c", subcore_axis_name="s"),
        scratch_types=[vm(_NLUT_PAD), vm(_NLUT_PAD), vm(_NLUT_PAD)]
                      + [vm(_CHUNK)] * 12
                      + [pltpu.SemaphoreType.DMA] * 5,
        compiler_params=pltpu.CompilerParams(needs_layout_passes=False),
    )
    return k(imgf, lutp).reshape(_B, _C, 512, 512)


# speedup vs baseline: 1.0149x; 1.0149x over previous
import jax
import jax.numpy as jnp
from jax import lax
from jax.experimental import pallas as pl
from jax.experimental.pallas import tpu as pltpu
from jax.experimental.pallas import tpu_sc as plsc

_B, _C = 4, 3
_GRID = 33
_NPIX = 512 * 512
_NLUT = _GRID * _GRID * _GRID
_NLUT_PAD = 35944
_NC, _NS = 2, 16
_NW = _NC * _NS
_TPB = _NW // _B
_PPT = _NPIX // _TPB
_CHUNK = 1024
_NCHUNK = _PPT // _CHUNK
_L = 16


_CMAX = 31.999998092651367
_SHIFTS = (0, 1, _GRID, _GRID + 1,
           _GRID * _GRID, _GRID * _GRID + 1,
           _GRID * _GRID + _GRID, _GRID * _GRID + _GRID + 1)

def _sc_body(img_hbm, lut_hbm, out_hbm,
             lut0, lut1, lut2,
             xa, ya, za, oa0, oa1, oa2,
             xb, yb, zb, ob0, ob1, ob2,
             sem_ia, sem_ib, sem_oa, sem_ob, sem_lut):
    wid = lax.axis_index("s") * _NC + lax.axis_index("c")
    b = wid // _TPB
    sub = wid % _TPB
    lbase = b * (_C * _NLUT_PAD)
    pltpu.async_copy(lut_hbm.at[pl.ds(pl.multiple_of(lbase, 8), _NLUT_PAD)], lut0, sem_lut)
    pltpu.async_copy(lut_hbm.at[pl.ds(pl.multiple_of(lbase + _NLUT_PAD, 8), _NLUT_PAD)], lut1, sem_lut)
    pltpu.async_copy(lut_hbm.at[pl.ds(pl.multiple_of(lbase + 2 * _NLUT_PAD, 8), _NLUT_PAD)], lut2, sem_lut)
    base = b * (_C * _NPIX) + sub * _PPT

    set_a = (xa, ya, za, oa0, oa1, oa2, sem_ia, sem_oa)
    set_b = (xb, yb, zb, ob0, ob1, ob2, sem_ib, sem_ob)

    def start_in(bufs, t):
        x, y, z, _, _, _, sem_i, _ = bufs
        off = pl.multiple_of(base + t * _CHUNK, _CHUNK)
        pltpu.async_copy(img_hbm.at[pl.ds(off, _CHUNK)], x, sem_i)
        pltpu.async_copy(img_hbm.at[pl.ds(off + _NPIX, _CHUNK)], y, sem_i)
        pltpu.async_copy(img_hbm.at[pl.ds(off + 2 * _NPIX, _CHUNK)], z, sem_i)

    def wait_in(bufs):
        x, y, z, _, _, _, sem_i, _ = bufs
        for d in (x, y, z):
            pltpu.make_async_copy(img_hbm.at[pl.ds(0, _CHUNK)], d, sem_i).wait()

    def start_out(bufs, t):
        _, _, _, p0, p1, p2, _, sem_o = bufs
        off = pl.multiple_of(base + t * _CHUNK, _CHUNK)
        pltpu.async_copy(p0, out_hbm.at[pl.ds(off, _CHUNK)], sem_o)
        pltpu.async_copy(p1, out_hbm.at[pl.ds(off + _NPIX, _CHUNK)], sem_o)
        pltpu.async_copy(p2, out_hbm.at[pl.ds(off + 2 * _NPIX, _CHUNK)], sem_o)

    def wait_out(bufs):
        _, _, _, p0, p1, p2, _, sem_o = bufs
        for s in (p0, p1, p2):
            pltpu.make_async_copy(s, out_hbm.at[pl.ds(0, _CHUNK)], sem_o).wait()

    def compute(bufs):
        x_r, y_r, z_r, p0, p1, p2, _, _ = bufs

        @plsc.parallel_loop(0, _CHUNK, step=_L, unroll=2)
        def _grp(g):
            s = pl.multiple_of(g, _L)

            def coord(v):
                cc = jnp.minimum(jnp.maximum(v * (_GRID - 1.0), 0.0), _CMAX)
                i0 = cc.astype(jnp.int32)
                w = cc - i0.astype(jnp.float32)
                return i0, w

            x0, wx = coord(x_r[pl.ds(s, _L)])
            y0, wy = coord(y_r[pl.ds(s, _L)])
            z0, wz = coord(z_r[pl.ds(s, _L)])
            i000 = (z0 * _GRID + y0) * _GRID + x0
            idx = [i000 + k if k else i000 for k in _SHIFTS]

            ux = 1.0 - wx
            w11 = wx * wy
            w10 = wy - w11
            w01 = wx - w11
            w00 = ux - w10

            corners = [[plsc.load_gather(ref, [idx[k]]) for k in range(8)]
                       for ref in (lut0, lut1, lut2)]
            for (c000, c001, c010, c011, c100, c101, c110, c111), ob in zip(
                    corners, (p0, p1, p2)):
                r0 = (w00 * c000 + w01 * c001) + (w10 * c010 + w11 * c011)
                r1 = (w00 * c100 + w01 * c101) + (w10 * c110 + w11 * c111)
                ob[pl.ds(s, _L)] = r0 + wz * (r1 - r0)

    start_in(set_a, 0)
    for d in (lut0, lut1, lut2):
        pltpu.make_async_copy(lut_hbm.at[pl.ds(0, _NLUT_PAD)], d, sem_lut).wait()

    @pl.loop(0, _NCHUNK, step=2)
    def _pair(t):
        start_in(set_b, t + 1)
        wait_in(set_a)

        @pl.when(t >= 2)
        def _():
            wait_out(set_a)

        compute(set_a)
        start_out(set_a, t)

        @pl.when(t + 2 < _NCHUNK)
        def _():
            start_in(set_a, t + 2)

        wait_in(set_b)

        @pl.when(t >= 2)
        def _():
            wait_out(set_b)

        compute(set_b)
        start_out(set_b, t + 1)

    wait_out(set_a)
    wait_out(set_b)


def kernel(img, lut):
    imgf = img.reshape(_B * _C * _NPIX)
    lutf = lut.reshape(_B, _C, _NLUT)
    lutp = jnp.pad(lutf, ((0, 0), (0, 0), (0, _NLUT_PAD - _NLUT)))
    lutp = lutp.reshape(_B * _C * _NLUT_PAD)
    vm = lambda n: pltpu.VMEM((n,), jnp.float32)
    k = pl.kernel(
        _sc_body,
        out_type=jax.ShapeDtypeStruct((_B * _C * _NPIX,), jnp.float32),
        mesh=plsc.VectorSubcoreMesh(core_axis_name="c", subcore_axis_name="s"),
        scratch_types=[vm(_NLUT_PAD), vm(_NLUT_PAD), vm(_NLUT_PAD)]
                      + [vm(_CHUNK)] * 12
                      + [pltpu.SemaphoreType.DMA] * 5,
        compiler_params=pltpu.CompilerParams(needs_layout_passes=False),
    )
    return k(imgf, lutp).reshape(_B, _C, 512, 512)
